# baseline (speedup 1.0000x reference)
import functools

import jax
import jax.numpy as jnp
from jax import lax
from jax.experimental import pallas as pl
from jax.experimental.pallas import tpu as pltpu
from jax.experimental.pallas import tpu_sc as plsc

B, CIN, C, N, K = 2, 256, 256, 2048, 16
TNA = 1024
TNC = 256
TNB = 256
ROWS = B * N * K
NRB = N * K
NW = 32
CH = 128


def _prep_body(xc_ref, pr_ref, pt_ref, wint_ref, bin_ref, g1_ref, be1_ref,
               wqt_ref, h_ref, tab_ref, qw_ref, idx_ref):
    q = pl.program_id(1)
    xc = xc_ref[0]
    h = lax.dot_general(xc, wint_ref[...], (((0,), (1,)), ((), ())),
                        preferred_element_type=jnp.float32)
    h = h + bin_ref[...]
    mu = jnp.mean(h, axis=1, keepdims=True)
    var = jnp.mean((h - mu) ** 2, axis=1, keepdims=True)
    hn = (h - mu) * lax.rsqrt(var + 1e-5) * g1_ref[...] + be1_ref[...]
    h_ref[0] = h
    p = pr_ref[0]
    b16 = lambda x: (lax.bitcast_convert_type(
        x.astype(jnp.bfloat16).astype(jnp.float32), jnp.int32)
        >> 16) & jnp.int32(0xFFFF)
    u = b16(hn)
    p_hi = p.astype(jnp.bfloat16).astype(jnp.float32)
    tab_ref[0, :, :PW] = u[:, :PW] | (u[:, PW:] << 16)
    tab_ref[0, :, PW:] = b16(p_hi) | (b16(p - p_hi) << 16)
    qw_ref[0] = jnp.dot(hn, wqt_ref[...], preferred_element_type=jnp.float32)

    PT = pt_ref[0]
    d = 2.0 * jnp.dot(p, PT, preferred_element_type=jnp.float32)
    d = d - jnp.sum(p * p, axis=1, keepdims=True)
    d = d - jnp.sum(PT * PT, axis=0, keepdims=True)
    col = lax.broadcasted_iota(jnp.int32, (TNA, N), 1)
    row_g = lax.broadcasted_iota(jnp.int32, (TNA, N), 0) + q * TNA
    d = jnp.where(col == row_g, -1e9, d)
    bits = lax.bitcast_convert_type(d, jnp.int32)
    sb = bits ^ ((bits >> 31) & jnp.int32(0x7FFFFFFF))
    pk = (sb & jnp.int32(~2047)) | (2047 - col)
    acc = jnp.zeros((TNA, K), jnp.int32)
    kcol = lax.broadcasted_iota(jnp.int32, (TNA, K), 1)
    int_min = jnp.int32(-(2 ** 31))
    for it in range(K):
        m = jnp.max(pk, axis=1, keepdims=True)
        pk = jnp.where(pk == m, int_min, pk)
        acc = jnp.where(kcol == it, 2047 - (m & 2047), acc)
    idx_ref[0] = acc


def _prep_call(xc, pr128, pt128, W_in, binr, g1r, be1r, wqt):
    f32 = jnp.float32
    return pl.pallas_call(
        _prep_body,
        grid=(1, N // TNA),
        in_specs=[
            pl.BlockSpec((1, CIN, TNA), lambda b, q: (b, 0, q)),
            pl.BlockSpec((1, TNA, PW), lambda b, q: (b, q, 0)),
            pl.BlockSpec((1, PW, N), lambda b, q: (b, 0, 0)),
            pl.BlockSpec((C, CIN), lambda b, q: (0, 0)),
            pl.BlockSpec((1, C), lambda b, q: (0, 0)),
            pl.BlockSpec((1, C), lambda b, q: (0, 0)),
            pl.BlockSpec((1, C), lambda b, q: (0, 0)),
            pl.BlockSpec((C, C), lambda b, q: (0, 0)),
        ],
        out_specs=[
            pl.BlockSpec((1, TNA, C), lambda b, q: (b, q, 0)),
            pl.BlockSpec((1, TNA, TW), lambda b, q: (b, q, 0)),
            pl.BlockSpec((1, TNA, C), lambda b, q: (b, q, 0)),
            pl.BlockSpec((1, TNA, K), lambda b, q: (b, q, 0)),
        ],
        out_shape=[
            jax.ShapeDtypeStruct((1, N, C), f32),
            jax.ShapeDtypeStruct((1, N, TW), jnp.int32),
            jax.ShapeDtypeStruct((1, N, C), f32),
            jax.ShapeDtypeStruct((1, N, K), jnp.int32),
        ],
    )(xc, pr128, pt128, W_in, binr, g1r, be1r, wqt)


PW = 128
TW = 256
NB = 2


def _gather_body(tab_hbm, idx_hbm, out_hbm, idx_v, rows_v, sem1, sem2):
    wid = lax.axis_index("s") * 2 + lax.axis_index("c")
    rows_per_w = NRB // NW
    sems = (sem1, sem2)

    def body(g, _):
        bases = []
        for bf in range(NB):
            base = wid * rows_per_w + (g * NB + bf) * CH
            bases.append(base)
            pltpu.sync_copy(idx_hbm.at[pl.ds(base, CH)], idx_v.at[bf])
            pltpu.make_async_copy(
                tab_hbm.at[idx_v.at[bf]], rows_v.at[bf], sems[bf]).start()
        for bf in range(NB):
            pltpu.make_async_copy(
                tab_hbm.at[idx_v.at[bf]], rows_v.at[bf], sems[bf]).wait()
            pltpu.sync_copy(rows_v.at[bf], out_hbm.at[pl.ds(bases[bf], CH)])
        return 0

    lax.fori_loop(0, rows_per_w // (CH * NB), body, 0)


def _sc_gather(tab_flat, idx_flat):
    mesh = plsc.VectorSubcoreMesh(core_axis_name="c", subcore_axis_name="s")
    run = functools.partial(
        pl.kernel,
        out_type=jax.ShapeDtypeStruct((NRB, TW), jnp.int32),
        mesh=mesh,
        scratch_types=[
            pltpu.VMEM((NB, CH), jnp.int32),
            pltpu.VMEM((NB, CH, TW), jnp.int32),
            pltpu.SemaphoreType.DMA,
            pltpu.SemaphoreType.DMA,
        ],
    )(_gather_body)
    return run(tab_flat, idx_flat)


def _attn_body(g_ref, pr_ref, qw_ref, h_ref,
               wd1_ref, bd1_ref, wd2_ref, bd2_ref, wal2_ref, bpr_ref,
               wk_ref, wv_ref, g2_ref, be2_ref,
               wf1_ref, bf1_ref, wf2_ref, bf2_ref, out_ref):
    TK = TNB * K
    bf16 = jnp.bfloat16
    dot16 = lambda x, w: jnp.dot(x.astype(bf16), w.astype(bf16),
                                 preferred_element_type=jnp.float32)
    w_hn = g_ref[:, :PW]
    w_p = g_ref[:, PW:]
    unlo = lambda w: lax.bitcast_convert_type(w << 16, jnp.float32)
    unhi = lambda w: lax.bitcast_convert_type(
        w & jnp.int32(-65536), jnp.float32)
    hj = jnp.concatenate([unlo(w_hn), unhi(w_hn)], axis=1)
    pj = unlo(w_p) + unhi(w_p)
    pi = pr_ref[0]

    diff = (pi.reshape(TNB, 1, PW) - pj.reshape(TNB, K, PW)).reshape(TK, PW)
    e = jnp.maximum(dot16(diff, wd1_ref[...]) + bd1_ref[...], 0.0)

    dpos = dot16(e, wd2_ref[...]) + bd2_ref[...]
    lpos = dot16(e, wal2_ref[...])
    kw = dot16(hj, wk_ref[...])
    v = dot16(hj, wv_ref[...])

    qw = qw_ref[0]

    l3 = (qw.reshape(TNB, 1, C) - kw.reshape(TNB, K, C)
          + lpos.reshape(TNB, K, C) + bpr_ref[...])
    val3 = (v + dpos).reshape(TNB, K, C)

    a = jnp.exp(l3)
    s = jnp.sum(a, axis=1)
    y = jnp.sum(a * val3, axis=1) / s

    hnew = h_ref[0] + y
    mu = jnp.mean(hnew, axis=1, keepdims=True)
    var = jnp.mean((hnew - mu) ** 2, axis=1, keepdims=True)
    h2 = (hnew - mu) * lax.rsqrt(var + 1e-5) * g2_ref[...] + be2_ref[...]
    f = jnp.maximum(dot16(h2, wf1_ref[...]) + bf1_ref[...], 0.0)
    res = hnew + dot16(f, wf2_ref[...]) + bf2_ref[...]
    out_ref[0] = jnp.transpose(res, (1, 0))


def _attn_call(g, pr16, qw, h,
               wd1p, bd1r, wd2t, bd2r, wal2t, bprr,
               wkt, wvt, g2r, be2r, wf1t, bf1r, wf2t, bf2r):
    TK = TNB * K
    C4 = 4 * C
    full = lambda shape: pl.BlockSpec(shape, lambda b, q: tuple(0 for _ in shape))
    return pl.pallas_call(
        _attn_body,
        grid=(1, N // TNB),
        in_specs=[
            pl.BlockSpec((TK, TW), lambda b, q: (b * (N // TNB) + q, 0)),
            pl.BlockSpec((1, TNB, PW), lambda b, q: (b, q, 0)),
            pl.BlockSpec((1, TNB, C), lambda b, q: (b, q, 0)),
            pl.BlockSpec((1, TNB, C), lambda b, q: (b, q, 0)),
            full((PW, C)), full((1, C)), full((C, C)), full((1, C)),
            full((C, C)), full((1, C)), full((C, C)), full((C, C)),
            full((1, C)), full((1, C)),
            full((C, C4)), full((1, C4)), full((C4, C)), full((1, C)),
        ],
        out_specs=pl.BlockSpec((1, C, TNB), lambda b, q: (b, 0, q)),
        out_shape=jax.ShapeDtypeStruct((1, C, N), jnp.float32),
    )(g, pr16, qw, h,
      wd1p, bd1r, wd2t, bd2r, wal2t, bprr,
      wkt, wvt, g2r, be2r, wf1t, bf1r, wf2t, bf2r)


def kernel(x, p, W_in, b_in, g1, be1, g2, be2, W_q, W_k, W_v,
           W_d1, b_d1, W_d2, b_d2, W_al, b_al, W_f1, b_f1, W_f2, b_f2):
    f32 = jnp.float32
    pr128 = jnp.pad(jnp.transpose(p, (0, 2, 1)), ((0, 0), (0, 0), (0, PW - 3)))
    pt128 = jnp.pad(p, ((0, 0), (0, PW - 3), (0, 0)))

    wqt = (W_al @ W_q).T
    wkt = (W_al @ W_k).T
    wvt = W_v.T
    wd1p = jnp.pad(W_d1.T, ((0, PW - 3), (0, 0)))
    wd2t = W_d2.T
    wal2t = (W_al @ W_d2).T
    bprr = (b_al + W_al @ b_d2).reshape(1, C)
    row = lambda v: v.reshape(1, -1).astype(f32)

    outs = []
    for b in range(B):
        h, tab, qw, idx = _prep_call(x[b:b + 1], pr128[b:b + 1],
                                     pt128[b:b + 1], W_in,
                                     row(b_in), row(g1), row(be1), wqt)
        g = _sc_gather(tab.reshape(N, TW), idx.reshape(NRB))
        outs.append(_attn_call(g, pr128[b:b + 1], qw, h,
                               wd1p, row(b_d1), wd2t, row(b_d2), wal2t, bprr,
                               wkt, wvt, row(g2), row(be2),
                               W_f1.T, row(b_f1), W_f2.T, row(b_f2)))
    return jnp.concatenate(outs, axis=0)


# speedup vs baseline: 1.0949x; 1.0949x over previous
import functools

import jax
import jax.numpy as jnp
from jax import lax
from jax.experimental import pallas as pl
from jax.experimental.pallas import tpu as pltpu
from jax.experimental.pallas import tpu_sc as plsc

B, CIN, C, N, K = 2, 256, 256, 2048, 16
TNA = 512
TNC = 256
TNB = 256
ROWS = B * N * K
NRB = N * K
NW = 32
CH = 128


def _prep_body(xc_ref, pr_ref, pt_ref, wint_ref, bin_ref, g1_ref, be1_ref,
               wqt_ref, h_ref, tab_ref, qw_ref, idx_ref):
    q = pl.program_id(1)
    xc = xc_ref[0]
    h = lax.dot_general(xc, wint_ref[...], (((0,), (1,)), ((), ())),
                        preferred_element_type=jnp.float32)
    h = h + bin_ref[...]
    mu = jnp.mean(h, axis=1, keepdims=True)
    var = jnp.mean((h - mu) ** 2, axis=1, keepdims=True)
    hn = (h - mu) * lax.rsqrt(var + 1e-5) * g1_ref[...] + be1_ref[...]
    h_ref[0] = h
    p = pr_ref[0]
    b16 = lambda x: (lax.bitcast_convert_type(
        x.astype(jnp.bfloat16).astype(jnp.float32), jnp.int32)
        >> 16) & jnp.int32(0xFFFF)
    u = b16(hn)
    p_hi = p.astype(jnp.bfloat16).astype(jnp.float32)
    tab_ref[0, :, :PW] = u[:, :PW] | (u[:, PW:] << 16)
    tab_ref[0, :, PW:] = b16(p_hi) | (b16(p - p_hi) << 16)
    qw_ref[0] = jnp.dot(hn, wqt_ref[...], preferred_element_type=jnp.float32)

    PT = pt_ref[0]
    d = 2.0 * jnp.dot(p, PT, preferred_element_type=jnp.float32)
    d = d - jnp.sum(p * p, axis=1, keepdims=True)
    d = d - jnp.sum(PT * PT, axis=0, keepdims=True)
    col = lax.broadcasted_iota(jnp.int32, (TNA, N), 1)
    row_g = lax.broadcasted_iota(jnp.int32, (TNA, N), 0) + q * TNA
    d = jnp.where(col == row_g, -1e9, d)
    bits = lax.bitcast_convert_type(d, jnp.int32)
    sb = bits ^ ((bits >> 31) & jnp.int32(0x7FFFFFFF))
    pk = (sb & jnp.int32(~2047)) | (2047 - col)
    acc = jnp.zeros((TNA, K), jnp.int32)
    kcol = lax.broadcasted_iota(jnp.int32, (TNA, K), 1)
    int_min = jnp.int32(-(2 ** 31))
    for it in range(K):
        m = jnp.max(pk, axis=1, keepdims=True)
        pk = jnp.where(pk == m, int_min, pk)
        acc = jnp.where(kcol == it, 2047 - (m & 2047), acc)
    idx_ref[0] = acc


def _prep_call(xc, pr128, pt128, W_in, binr, g1r, be1r, wqt):
    f32 = jnp.float32
    return pl.pallas_call(
        _prep_body,
        grid=(1, N // TNA),
        in_specs=[
            pl.BlockSpec((1, CIN, TNA), lambda b, q: (b, 0, q)),
            pl.BlockSpec((1, TNA, PW), lambda b, q: (b, q, 0)),
            pl.BlockSpec((1, PW, N), lambda b, q: (b, 0, 0)),
            pl.BlockSpec((C, CIN), lambda b, q: (0, 0)),
            pl.BlockSpec((1, C), lambda b, q: (0, 0)),
            pl.BlockSpec((1, C), lambda b, q: (0, 0)),
            pl.BlockSpec((1, C), lambda b, q: (0, 0)),
            pl.BlockSpec((C, C), lambda b, q: (0, 0)),
        ],
        out_specs=[
            pl.BlockSpec((1, TNA, C), lambda b, q: (b, q, 0)),
            pl.BlockSpec((1, TNA, TW), lambda b, q: (b, q, 0)),
            pl.BlockSpec((1, TNA, C), lambda b, q: (b, q, 0)),
            pl.BlockSpec((1, TNA, K), lambda b, q: (b, q, 0)),
        ],
        out_shape=[
            jax.ShapeDtypeStruct((1, N, C), f32),
            jax.ShapeDtypeStruct((1, N, TW), jnp.int32),
            jax.ShapeDtypeStruct((1, N, C), f32),
            jax.ShapeDtypeStruct((1, N, K), jnp.int32),
        ],
    )(xc, pr128, pt128, W_in, binr, g1r, be1r, wqt)


PW = 128
TW = 256
NB = 2


def _gather_body(tab_hbm, idx_hbm, out_hbm, idx_v, rows_v, sem1, sem2):
    wid = lax.axis_index("s") * 2 + lax.axis_index("c")
    rows_per_w = NRB // NW
    sems = (sem1, sem2)

    def body(g, _):
        bases = []
        for bf in range(NB):
            base = wid * rows_per_w + (g * NB + bf) * CH
            bases.append(base)
            pltpu.sync_copy(idx_hbm.at[pl.ds(base, CH)], idx_v.at[bf])
            pltpu.make_async_copy(
                tab_hbm.at[idx_v.at[bf]], rows_v.at[bf], sems[bf]).start()
        for bf in range(NB):
            pltpu.make_async_copy(
                tab_hbm.at[idx_v.at[bf]], rows_v.at[bf], sems[bf]).wait()
            pltpu.sync_copy(rows_v.at[bf], out_hbm.at[pl.ds(bases[bf], CH)])
        return 0

    lax.fori_loop(0, rows_per_w // (CH * NB), body, 0)


def _sc_gather(tab_flat, idx_flat):
    mesh = plsc.VectorSubcoreMesh(core_axis_name="c", subcore_axis_name="s")
    run = functools.partial(
        pl.kernel,
        out_type=jax.ShapeDtypeStruct((NRB, TW), jnp.int32),
        mesh=mesh,
        scratch_types=[
            pltpu.VMEM((NB, CH), jnp.int32),
            pltpu.VMEM((NB, CH, TW), jnp.int32),
            pltpu.SemaphoreType.DMA,
            pltpu.SemaphoreType.DMA,
        ],
    )(_gather_body)
    return run(tab_flat, idx_flat)


def _attn_body(g_ref, pr_ref, qw_ref, h_ref,
               wd1_ref, bd1_ref, wd2_ref, bd2_ref, wal2_ref, bpr_ref,
               wk_ref, wv_ref, g2_ref, be2_ref,
               wf1_ref, bf1_ref, wf2_ref, bf2_ref, out_ref):
    TK = TNB * K
    bf16 = jnp.bfloat16
    dot16 = lambda x, w: jnp.dot(x.astype(bf16), w.astype(bf16),
                                 preferred_element_type=jnp.float32)
    w_hn = g_ref[:, :PW]
    w_p = g_ref[:, PW:]
    unlo = lambda w: lax.bitcast_convert_type(w << 16, jnp.float32)
    unhi = lambda w: lax.bitcast_convert_type(
        w & jnp.int32(-65536), jnp.float32)
    hj = jnp.concatenate([unlo(w_hn), unhi(w_hn)], axis=1)
    pj = unlo(w_p) + unhi(w_p)
    pi = pr_ref[0]

    diff = (pi.reshape(TNB, 1, PW) - pj.reshape(TNB, K, PW)).reshape(TK, PW)
    e = jnp.maximum(dot16(diff, wd1_ref[...]) + bd1_ref[...], 0.0)

    dpos = dot16(e, wd2_ref[...]) + bd2_ref[...]
    lpos = dot16(e, wal2_ref[...])
    kw = dot16(hj, wk_ref[...])
    v = dot16(hj, wv_ref[...])

    qw = qw_ref[0]

    l3 = (qw.reshape(TNB, 1, C) - kw.reshape(TNB, K, C)
          + lpos.reshape(TNB, K, C) + bpr_ref[...])
    val3 = (v + dpos).reshape(TNB, K, C)

    a = jnp.exp(l3)
    s = jnp.sum(a, axis=1)
    y = jnp.sum(a * val3, axis=1) / s

    hnew = h_ref[0] + y
    mu = jnp.mean(hnew, axis=1, keepdims=True)
    var = jnp.mean((hnew - mu) ** 2, axis=1, keepdims=True)
    h2 = (hnew - mu) * lax.rsqrt(var + 1e-5) * g2_ref[...] + be2_ref[...]
    f = jnp.maximum(dot16(h2, wf1_ref[...]) + bf1_ref[...], 0.0)
    res = hnew + dot16(f, wf2_ref[...]) + bf2_ref[...]
    out_ref[0] = jnp.transpose(res, (1, 0))


def _attn_call(g, pr16, qw, h,
               wd1p, bd1r, wd2t, bd2r, wal2t, bprr,
               wkt, wvt, g2r, be2r, wf1t, bf1r, wf2t, bf2r):
    TK = TNB * K
    C4 = 4 * C
    full = lambda shape: pl.BlockSpec(shape, lambda b, q: tuple(0 for _ in shape))
    return pl.pallas_call(
        _attn_body,
        grid=(1, N // TNB),
        in_specs=[
            pl.BlockSpec((TK, TW), lambda b, q: (b * (N // TNB) + q, 0)),
            pl.BlockSpec((1, TNB, PW), lambda b, q: (b, q, 0)),
            pl.BlockSpec((1, TNB, C), lambda b, q: (b, q, 0)),
            pl.BlockSpec((1, TNB, C), lambda b, q: (b, q, 0)),
            full((PW, C)), full((1, C)), full((C, C)), full((1, C)),
            full((C, C)), full((1, C)), full((C, C)), full((C, C)),
            full((1, C)), full((1, C)),
            full((C, C4)), full((1, C4)), full((C4, C)), full((1, C)),
        ],
        out_specs=pl.BlockSpec((1, C, TNB), lambda b, q: (b, 0, q)),
        out_shape=jax.ShapeDtypeStruct((1, C, N), jnp.float32),
    )(g, pr16, qw, h,
      wd1p, bd1r, wd2t, bd2r, wal2t, bprr,
      wkt, wvt, g2r, be2r, wf1t, bf1r, wf2t, bf2r)


def kernel(x, p, W_in, b_in, g1, be1, g2, be2, W_q, W_k, W_v,
           W_d1, b_d1, W_d2, b_d2, W_al, b_al, W_f1, b_f1, W_f2, b_f2):
    f32 = jnp.float32
    pr128 = jnp.pad(jnp.transpose(p, (0, 2, 1)), ((0, 0), (0, 0), (0, PW - 3)))
    pt128 = jnp.pad(p, ((0, 0), (0, PW - 3), (0, 0)))

    wqt = (W_al @ W_q).T
    wkt = (W_al @ W_k).T
    wvt = W_v.T
    wd1p = jnp.pad(W_d1.T, ((0, PW - 3), (0, 0)))
    wd2t = W_d2.T
    wal2t = (W_al @ W_d2).T
    bprr = (b_al + W_al @ b_d2).reshape(1, C)
    row = lambda v: v.reshape(1, -1).astype(f32)

    outs = []
    for b in range(B):
        h, tab, qw, idx = _prep_call(x[b:b + 1], pr128[b:b + 1],
                                     pt128[b:b + 1], W_in,
                                     row(b_in), row(g1), row(be1), wqt)
        g = _sc_gather(tab.reshape(N, TW), idx.reshape(NRB))
        outs.append(_attn_call(g, pr128[b:b + 1], qw, h,
                               wd1p, row(b_d1), wd2t, row(b_d2), wal2t, bprr,
                               wkt, wvt, row(g2), row(be2),
                               W_f1.T, row(b_f1), W_f2.T, row(b_f2)))
    return jnp.concatenate(outs, axis=0)


# speedup vs baseline: 1.1226x; 1.0253x over previous
import functools

import jax
import jax.numpy as jnp
from jax import lax
from jax.experimental import pallas as pl
from jax.experimental.pallas import tpu as pltpu
from jax.experimental.pallas import tpu_sc as plsc

B, CIN, C, N, K = 2, 256, 256, 2048, 16
TNA = 512
TNC = 256
TNB = 256
ROWS = B * N * K
NRB = N * K
NW = 32
CH = 128


def _prep_body(xc_ref, pr_ref, pt_ref, wint_ref, bin_ref, g1_ref, be1_ref,
               wqt_ref, h_ref, tab_ref, qw_ref, idx_ref):
    q = pl.program_id(1)
    xc = xc_ref[0]
    h = lax.dot_general(xc, wint_ref[...], (((0,), (1,)), ((), ())),
                        preferred_element_type=jnp.float32)
    h = h + bin_ref[...]
    mu = jnp.mean(h, axis=1, keepdims=True)
    var = jnp.mean((h - mu) ** 2, axis=1, keepdims=True)
    hn = (h - mu) * lax.rsqrt(var + 1e-5) * g1_ref[...] + be1_ref[...]
    h_ref[0] = h
    p = pr_ref[0]
    b16 = lambda x: (lax.bitcast_convert_type(
        x.astype(jnp.bfloat16).astype(jnp.float32), jnp.int32)
        >> 16) & jnp.int32(0xFFFF)
    u = b16(hn)
    p_hi = p.astype(jnp.bfloat16).astype(jnp.float32)
    tab_ref[0, :, :PW] = u[:, :PW] | (u[:, PW:] << 16)
    tab_ref[0, :, PW:] = b16(p_hi) | (b16(p - p_hi) << 16)
    qw_ref[0] = jnp.dot(hn, wqt_ref[...], preferred_element_type=jnp.float32)

    PT = pt_ref[0]
    d = 2.0 * jnp.dot(p, PT, preferred_element_type=jnp.float32)
    d = d - jnp.sum(p * p, axis=1, keepdims=True)
    d = d - jnp.sum(PT * PT, axis=0, keepdims=True)
    col = lax.broadcasted_iota(jnp.int32, (TNA, N), 1)
    row_g = lax.broadcasted_iota(jnp.int32, (TNA, N), 0) + q * TNA
    d = jnp.where(col == row_g, -1e9, d)
    bits = lax.bitcast_convert_type(d, jnp.int32)
    sb = bits ^ ((bits >> 31) & jnp.int32(0x7FFFFFFF))
    pk = (sb & jnp.int32(~2047)) | (2047 - col)
    acc = jnp.zeros((TNA, K), jnp.int32)
    kcol = lax.broadcasted_iota(jnp.int32, (TNA, K), 1)
    int_min = jnp.int32(-(2 ** 31))
    for it in range(K):
        m = jnp.max(pk, axis=1, keepdims=True)
        pk = jnp.where(pk == m, int_min, pk)
        acc = jnp.where(kcol == it, 2047 - (m & 2047), acc)
    idx_ref[0] = acc


def _prep_call(xc, pr128, pt128, W_in, binr, g1r, be1r, wqt):
    f32 = jnp.float32
    return pl.pallas_call(
        _prep_body,
        grid=(1, N // TNA),
        in_specs=[
            pl.BlockSpec((1, CIN, TNA), lambda b, q: (b, 0, q)),
            pl.BlockSpec((1, TNA, PW), lambda b, q: (b, q, 0)),
            pl.BlockSpec((1, PW, N), lambda b, q: (b, 0, 0)),
            pl.BlockSpec((C, CIN), lambda b, q: (0, 0)),
            pl.BlockSpec((1, C), lambda b, q: (0, 0)),
            pl.BlockSpec((1, C), lambda b, q: (0, 0)),
            pl.BlockSpec((1, C), lambda b, q: (0, 0)),
            pl.BlockSpec((C, C), lambda b, q: (0, 0)),
        ],
        out_specs=[
            pl.BlockSpec((1, TNA, C), lambda b, q: (b, q, 0)),
            pl.BlockSpec((1, TNA, TW), lambda b, q: (b, q, 0)),
            pl.BlockSpec((1, TNA, C), lambda b, q: (b, q, 0)),
            pl.BlockSpec((1, TNA, K), lambda b, q: (b, q, 0)),
        ],
        out_shape=[
            jax.ShapeDtypeStruct((1, N, C), f32),
            jax.ShapeDtypeStruct((1, N, TW), jnp.int32),
            jax.ShapeDtypeStruct((1, N, C), f32),
            jax.ShapeDtypeStruct((1, N, K), jnp.int32),
        ],
    )(xc, pr128, pt128, W_in, binr, g1r, be1r, wqt)


PW = 128
TW = 256
NB = 2


def _gather_body(tab_hbm, idx_hbm, out_hbm, idx_v, rows_v, sem1, sem2):
    wid = lax.axis_index("s") * 2 + lax.axis_index("c")
    rows_per_w = NRB // NW
    sems = (sem1, sem2)

    def body(g, _):
        bases = []
        for bf in range(NB):
            base = wid * rows_per_w + (g * NB + bf) * CH
            bases.append(base)
            pltpu.sync_copy(idx_hbm.at[pl.ds(base, CH)], idx_v.at[bf])
            pltpu.make_async_copy(
                tab_hbm.at[idx_v.at[bf]], rows_v.at[bf], sems[bf]).start()
        for bf in range(NB):
            pltpu.make_async_copy(
                tab_hbm.at[idx_v.at[bf]], rows_v.at[bf], sems[bf]).wait()
            pltpu.sync_copy(rows_v.at[bf], out_hbm.at[pl.ds(bases[bf], CH)])
        return 0

    lax.fori_loop(0, rows_per_w // (CH * NB), body, 0)


def _sc_gather(tab_flat, idx_flat):
    mesh = plsc.VectorSubcoreMesh(core_axis_name="c", subcore_axis_name="s")
    run = functools.partial(
        pl.kernel,
        out_type=jax.ShapeDtypeStruct((NRB, TW), jnp.int32),
        mesh=mesh,
        scratch_types=[
            pltpu.VMEM((NB, CH), jnp.int32),
            pltpu.VMEM((NB, CH, TW), jnp.int32),
            pltpu.SemaphoreType.DMA,
            pltpu.SemaphoreType.DMA,
        ],
    )(_gather_body)
    return run(tab_flat, idx_flat)


def _attn_body(g_ref, pr_ref, qw_ref, h_ref,
               wd1_ref, bd1_ref, wd2_ref, bd2_ref, bpr_ref,
               wk_ref, g2_ref, be2_ref,
               wf1_ref, bf1_ref, wf2_ref, bf2_ref, out_ref):
    TK = TNB * K
    bf16 = jnp.bfloat16
    dot16 = lambda x, w: jnp.dot(x.astype(bf16), w.astype(bf16),
                                 preferred_element_type=jnp.float32)
    w_hn = g_ref[:, :PW]
    w_p = g_ref[:, PW:]
    unlo = lambda w: lax.bitcast_convert_type(w << 16, jnp.float32)
    unhi = lambda w: lax.bitcast_convert_type(
        w & jnp.int32(-65536), jnp.float32)
    hj = jnp.concatenate([unlo(w_hn), unhi(w_hn)], axis=1)
    pj = unlo(w_p) + unhi(w_p)
    pi = pr_ref[0]

    diff = (pi.reshape(TNB, 1, PW) - pj.reshape(TNB, K, PW)).reshape(TK, PW)
    e = jnp.maximum(dot16(diff, wd1_ref[...]) + bd1_ref[...], 0.0)

    dl = dot16(e, wd2_ref[...])
    dpos = dl[:, :C] + bd2_ref[...]
    lpos = dl[:, C:]
    kv = dot16(hj, wk_ref[...])
    kw = kv[:, :C]
    v = kv[:, C:]

    qw = qw_ref[0]

    l3 = (qw.reshape(TNB, 1, C) - kw.reshape(TNB, K, C)
          + lpos.reshape(TNB, K, C) + bpr_ref[...])
    val3 = (v + dpos).reshape(TNB, K, C)

    a = jnp.exp(l3)
    s = jnp.sum(a, axis=1)
    y = jnp.sum(a * val3, axis=1) / s

    hnew = h_ref[0] + y
    mu = jnp.mean(hnew, axis=1, keepdims=True)
    var = jnp.mean((hnew - mu) ** 2, axis=1, keepdims=True)
    h2 = (hnew - mu) * lax.rsqrt(var + 1e-5) * g2_ref[...] + be2_ref[...]
    f = jnp.maximum(dot16(h2, wf1_ref[...]) + bf1_ref[...], 0.0)
    res = hnew + dot16(f, wf2_ref[...]) + bf2_ref[...]
    out_ref[0] = jnp.transpose(res, (1, 0))


def _attn_call(g, pr16, qw, h,
               wd1p, bd1r, wdlt, bd2r, bprr,
               wkvt, g2r, be2r, wf1t, bf1r, wf2t, bf2r):
    TK = TNB * K
    C4 = 4 * C
    full = lambda shape: pl.BlockSpec(shape, lambda b, q: tuple(0 for _ in shape))
    return pl.pallas_call(
        _attn_body,
        grid=(1, N // TNB),
        in_specs=[
            pl.BlockSpec((TK, TW), lambda b, q: (b * (N // TNB) + q, 0)),
            pl.BlockSpec((1, TNB, PW), lambda b, q: (b, q, 0)),
            pl.BlockSpec((1, TNB, C), lambda b, q: (b, q, 0)),
            pl.BlockSpec((1, TNB, C), lambda b, q: (b, q, 0)),
            full((PW, C)), full((1, C)), full((C, 2 * C)), full((1, C)),
            full((1, C)), full((C, 2 * C)),
            full((1, C)), full((1, C)),
            full((C, C4)), full((1, C4)), full((C4, C)), full((1, C)),
        ],
        out_specs=pl.BlockSpec((1, C, TNB), lambda b, q: (b, 0, q)),
        out_shape=jax.ShapeDtypeStruct((1, C, N), jnp.float32),
    )(g, pr16, qw, h,
      wd1p, bd1r, wdlt, bd2r, bprr,
      wkvt, g2r, be2r, wf1t, bf1r, wf2t, bf2r)


def kernel(x, p, W_in, b_in, g1, be1, g2, be2, W_q, W_k, W_v,
           W_d1, b_d1, W_d2, b_d2, W_al, b_al, W_f1, b_f1, W_f2, b_f2):
    f32 = jnp.float32
    pr128 = jnp.pad(jnp.transpose(p, (0, 2, 1)), ((0, 0), (0, 0), (0, PW - 3)))
    pt128 = jnp.pad(p, ((0, 0), (0, PW - 3), (0, 0)))

    wqt = (W_al @ W_q).T
    wkvt = jnp.concatenate([(W_al @ W_k).T, W_v.T], axis=1)
    wd1p = jnp.pad(W_d1.T, ((0, PW - 3), (0, 0)))
    wdlt = jnp.concatenate([W_d2.T, (W_al @ W_d2).T], axis=1)
    bprr = (b_al + W_al @ b_d2).reshape(1, C)
    row = lambda v: v.reshape(1, -1).astype(f32)

    outs = []
    for b in range(B):
        h, tab, qw, idx = _prep_call(x[b:b + 1], pr128[b:b + 1],
                                     pt128[b:b + 1], W_in,
                                     row(b_in), row(g1), row(be1), wqt)
        g = _sc_gather(tab.reshape(N, TW), idx.reshape(NRB))
        outs.append(_attn_call(g, pr128[b:b + 1], qw, h,
                               wd1p, row(b_d1), wdlt, row(b_d2), bprr,
                               wkvt, row(g2), row(be2),
                               W_f1.T, row(b_f1), W_f2.T, row(b_f2)))
    return jnp.concatenate(outs, axis=0)
